# baseline (speedup 1.0000x reference)
import jax
import jax.numpy as jnp
from jax.experimental import pallas as pl
from jax.experimental.pallas import tpu as pltpu

N = 1024
DIN = 64
DIM = 16
AH = 8
PH = 8
BLK_I = 128
GRP = 16
SB = BLK_I // GRP
NG = N // GRP


def _fused_kernel(feat, pos, featg, posg, W1, b1, Wq, bq, Wv, bv, Wp1,
                  Wa1, ba1, G1, b1g, Gk, bkg, Ga1, Gp1,
                  BDp2, bp2t, BDa1, BDa2, ba2t, bp1t, BDw2, b2g, out,
                  ppjt_s, qajt_s, vt_s, ppg_s, kag_s):
    pid = pl.program_id(0)

    @pl.when(pid == 0)
    def _proj():
        fT = jax.nn.relu(
            jax.lax.dot_general(W1[...], feat[...], (((0,), (1,)), ((), ())),
                                preferred_element_type=jnp.float32) + b1[...])
        tdot = lambda w, x: jax.lax.dot_general(
            w, x, (((0,), (0,)), ((), ())),
            preferred_element_type=jnp.float32)
        qT = jax.nn.relu(tdot(Wq[...], fT) + bq[...])
        vT = jax.nn.relu(tdot(Wv[...], fT) + bv[...])
        ppT = jax.lax.dot_general(Wp1[...], pos[...], (((0,), (1,)), ((), ())),
                                  preferred_element_type=jnp.float32)
        qaT = tdot(Wa1[...], qT) + ba1[...]
        ppjt_s[...] = jnp.concatenate([ppT] * GRP, axis=0)
        qajt_s[...] = jnp.concatenate([qaT] * GRP, axis=0)
        vt_s[...] = jnp.concatenate([vT] * GRP, axis=0)
        fg = jax.nn.relu(jnp.dot(featg[...], G1[...],
                                 preferred_element_type=jnp.float32) + b1g[...])
        kg = jax.nn.relu(jnp.dot(fg, Gk[...],
                                 preferred_element_type=jnp.float32) + bkg[...])
        kag_s[...] = jnp.dot(kg, Ga1[...], preferred_element_type=jnp.float32)
        ppg_s[...] = jnp.dot(posg[...], Gp1[...],
                             preferred_element_type=jnp.float32)

    ppit = ppg_s[pl.ds(pid * SB, SB), :][:, :, None]
    kait = kag_s[pl.ds(pid * SB, SB), :][:, :, None]
    ppjt = ppjt_s[...][None, :, :]
    qajt = qajt_s[...][None, :, :]

    def bdot(bd, x):
        bdb = jnp.broadcast_to(bd[None, :, :], (SB,) + bd.shape)
        return jax.lax.dot_general(
            bdb, x, (((2,), (1,)), ((0,), (0,))),
            preferred_element_type=jnp.float32)

    a = jax.nn.relu(ppjt - ppit + bp1t[...][None, :, :])
    pe = jax.nn.relu(bdot(BDp2[...], a) + bp2t[...][None, :, :])
    u = jax.nn.relu(bdot(BDa1[...], pe) + qajt - kait)
    e = jax.nn.relu(bdot(BDa2[...], u) + ba2t[...][None, :, :])
    w = jnp.exp(e)
    s = jnp.sum(w, axis=2, keepdims=True)
    o = jnp.sum(w * vt_s[...][None, :, :], axis=2, keepdims=True) / s
    o2 = o[:, :, 0]
    out[...] = jnp.dot(o2, BDw2[...],
                       preferred_element_type=jnp.float32) + b2g[...]


def kernel(feature, pos, W1, b1, Wq, bq, Wk, bk, Wv, bv,
           Wp1, bp1, Wp2, bp2, Wa1, ba1, Wa2, ba2, W2, b2):
    feat2 = feature.reshape(N, DIN)
    pos2 = pos.reshape(N, 3)
    featg = feature.reshape(NG, GRP * DIN)
    posg = pos.reshape(NG, GRP * 3)
    c = lambda x: x.reshape(-1, 1)
    r = lambda x: x.reshape(1, -1)
    eye = jnp.eye(GRP, dtype=jnp.float32)
    kr = lambda wst: jnp.kron(eye, wst)
    tc = lambda x: jnp.tile(c(x), (GRP, 1))
    tr = lambda x: jnp.tile(r(x), (1, GRP))

    grid = (N // BLK_I,)
    full = lambda shape: pl.BlockSpec(shape, lambda i: tuple(0 for _ in shape))
    out = pl.pallas_call(
        _fused_kernel,
        grid=grid,
        in_specs=[
            full((N, DIN)), full((N, 3)),
            full((NG, GRP * DIN)), full((NG, GRP * 3)),
            full((DIN, DIM)), full((DIM, 1)),
            full((DIM, DIM)), full((DIM, 1)),
            full((DIM, DIM)), full((DIM, 1)),
            full((3, PH)),
            full((DIM, AH)), full((AH, 1)),
            full((GRP * DIN, GRP * DIM)), full((1, GRP * DIM)),
            full((GRP * DIM, GRP * DIM)), full((1, GRP * DIM)),
            full((GRP * DIM, GRP * AH)), full((GRP * 3, GRP * PH)),
            full((GRP * DIM, GRP * PH)), full((GRP * DIM, 1)),
            full((GRP * AH, GRP * DIM)), full((GRP * DIM, GRP * AH)),
            full((GRP * DIM, 1)), full((GRP * PH, 1)),
            full((GRP * DIM, GRP * DIM)), full((1, GRP * DIM)),
        ],
        out_specs=pl.BlockSpec((SB, GRP * DIM), lambda i: (i, 0)),
        out_shape=jax.ShapeDtypeStruct((NG, GRP * DIM), jnp.float32),
        scratch_shapes=[
            pltpu.VMEM((GRP * PH, N), jnp.float32),
            pltpu.VMEM((GRP * AH, N), jnp.float32),
            pltpu.VMEM((GRP * DIM, N), jnp.float32),
            pltpu.VMEM((NG, GRP * PH), jnp.float32),
            pltpu.VMEM((NG, GRP * AH), jnp.float32),
        ],
        compiler_params=pltpu.CompilerParams(
            dimension_semantics=("arbitrary",)),
    )(feat2, pos2, featg, posg, W1, c(b1), Wq, c(bq), Wv, c(bv), Wp1,
      Wa1, c(ba1),
      kr(W1), tr(b1), kr(Wk), tr(bk), kr(Wa1), kr(Wp1),
      kr(Wp2.T), tc(bp2), kr(Wa1.T), kr(Wa2.T), tc(ba2), tc(bp1),
      kr(W2), tr(b2))

    return out.reshape(1, N, DIM)


# speedup vs baseline: 1.4160x; 1.4160x over previous
import jax
import jax.numpy as jnp
from jax.experimental import pallas as pl
from jax.experimental.pallas import tpu as pltpu

N = 1024
DIN = 64
DIM = 16
AH = 8
PH = 8
BLK_I = 256


def _fused_kernel(feat, pos, W1, b1, Wq, bq, Wk, bk, Wv, bv, Wp1, Wa1, ba1,
                  bp1, Wp2, bp2, Wa2, ba2, W2, b2, out,
                  ppT_s, qaT_s, vT_s, ppr_s, kar_s):
    pid = pl.program_id(0)

    @pl.when(pid == 0)
    def _proj():
        fT = jax.nn.relu(
            jax.lax.dot_general(W1[...], feat[...], (((0,), (1,)), ((), ())),
                                preferred_element_type=jnp.float32) + b1[...])
        tdot = lambda w, x: jax.lax.dot_general(
            w, x, (((0,), (0,)), ((), ())),
            preferred_element_type=jnp.float32)
        qT = jax.nn.relu(tdot(Wq[...], fT) + bq[...])
        kT = jax.nn.relu(tdot(Wk[...], fT) + bk[...])
        vT_s[...] = jax.nn.relu(tdot(Wv[...], fT) + bv[...])
        ppT = jax.lax.dot_general(Wp1[...], pos[...], (((0,), (1,)), ((), ())),
                                  preferred_element_type=jnp.float32)
        kaT = tdot(Wa1[...], kT)
        ppT_s[...] = ppT
        qaT_s[...] = tdot(Wa1[...], qT) + ba1[...]
        ppr_s[...] = ppT.T
        kar_s[...] = kaT.T

    i0 = pid * BLK_I
    ppi = ppr_s[pl.ds(i0, BLK_I), :][:, :, None]
    kai = kar_s[pl.ds(i0, BLK_I), :][:, :, None]
    ppj = ppT_s[...][None, :, :]
    qaj = qaT_s[...][None, :, :]

    def bdot(w, x):
        wb = jnp.broadcast_to(w[None, :, :], (BLK_I,) + w.shape)
        return jax.lax.dot_general(
            wb, x, (((2,), (1,)), ((0,), (0,))),
            preferred_element_type=jnp.float32)

    a = jax.nn.relu(ppj - ppi + bp1[...][None, :, :])
    pe = jax.nn.relu(bdot(Wp2[...].T, a) + bp2[...][None, :, :])
    u = jax.nn.relu(bdot(Wa1[...].T, pe) + qaj - kai)
    e = jax.nn.relu(bdot(Wa2[...].T, u) + ba2[...][None, :, :])
    w = jnp.exp(e)
    s = jnp.sum(w, axis=2, keepdims=True)
    o = jnp.sum(w * vT_s[...][None, :, :], axis=2, keepdims=True) / s
    o = o.reshape(BLK_I, DIM)
    out[...] = jnp.dot(o, W2[...], preferred_element_type=jnp.float32) + b2[...]


def kernel(feature, pos, W1, b1, Wq, bq, Wk, bk, Wv, bv,
           Wp1, bp1, Wp2, bp2, Wa1, ba1, Wa2, ba2, W2, b2):
    feat2 = feature.reshape(N, DIN)
    pos2 = pos.reshape(N, 3)
    c = lambda x: x.reshape(-1, 1)

    grid = (N // BLK_I,)
    full = lambda shape: pl.BlockSpec(shape, lambda i: tuple(0 for _ in shape))
    out = pl.pallas_call(
        _fused_kernel,
        grid=grid,
        in_specs=[
            full((N, DIN)), full((N, 3)),
            full((DIN, DIM)), full((DIM, 1)),
            full((DIM, DIM)), full((DIM, 1)),
            full((DIM, DIM)), full((DIM, 1)),
            full((DIM, DIM)), full((DIM, 1)),
            full((3, PH)), full((DIM, AH)), full((AH, 1)),
            full((PH, 1)), full((PH, DIM)), full((DIM, 1)),
            full((AH, DIM)), full((DIM, 1)),
            full((DIM, DIM)), full((1, DIM)),
        ],
        out_specs=pl.BlockSpec((BLK_I, DIM), lambda i: (i, 0)),
        out_shape=jax.ShapeDtypeStruct((N, DIM), jnp.float32),
        scratch_shapes=[
            pltpu.VMEM((PH, N), jnp.float32),
            pltpu.VMEM((AH, N), jnp.float32),
            pltpu.VMEM((DIM, N), jnp.float32),
            pltpu.VMEM((N, PH), jnp.float32),
            pltpu.VMEM((N, AH), jnp.float32),
        ],
        compiler_params=pltpu.CompilerParams(
            dimension_semantics=("arbitrary",)),
    )(feat2, pos2, W1, c(b1), Wq, c(bq), Wk, c(bk), Wv, c(bv),
      Wp1, Wa1, c(ba1), c(bp1), Wp2, c(bp2), Wa2, c(ba2), W2,
      b2.reshape(1, DIM))

    return out.reshape(1, N, DIM)
